# baseline (speedup 1.0000x reference)
import functools

import jax
import jax.numpy as jnp
from jax.experimental import pallas as pl
from jax.experimental.pallas import tpu as pltpu

_LOG2E = 1.4426950408889634
_CLAMP = 104.0


def _fused_kernel(ns_blk_ref, ns_nxt_ref, adj_ref, ns_ref, hid_ref, wq_ref,
                  wk_ref, wv_ref, w1_ref, b1_ref, w2_ref, b2_ref, step_ref,
                  out_ref, q_ref, s_ref, acct_ref):
    j = pl.program_id(0)
    nsteps = pl.num_programs(0)

    @pl.when(j == 0)
    def _prologue():
        q_ref[...] = jax.lax.dot_general(
            ns_ref[...], wq_ref[...], (((1,), (1,)), ((), ())),
            preferred_element_type=jnp.float32).astype(jnp.bfloat16)
        k0 = jax.lax.dot_general(
            ns_blk_ref[...], wk_ref[...], (((1,), (1,)), ((), ())),
            preferred_element_type=jnp.float32).astype(jnp.bfloat16)
        s_ref[0] = jax.lax.dot_general(
            k0, q_ref[...], (((1,), (1,)), ((), ())),
            preferred_element_type=jnp.float32).astype(jnp.bfloat16)

    ns_blk = ns_blk_ref[...]
    v_blk = jax.lax.dot_general(
        ns_blk, wv_ref[...], (((1,), (1,)), ((), ())),
        preferred_element_type=jnp.float32).astype(jnp.bfloat16)
    ones = jnp.ones((8, v_blk.shape[0]), dtype=jnp.bfloat16)
    vt_blk = jnp.concatenate([v_blk.T, ones], axis=0)

    @pl.when(j != nsteps - 1)
    def _next_scores():
        k_nxt = jax.lax.dot_general(
            ns_nxt_ref[...], wk_ref[...], (((1,), (1,)), ((), ())),
            preferred_element_type=jnp.float32).astype(jnp.bfloat16)
        s_ref[(j + 1) % 2] = jax.lax.dot_general(
            k_nxt, q_ref[...], (((1,), (1,)), ((), ())),
            preferred_element_type=jnp.float32).astype(jnp.bfloat16)

    s = s_ref[j % 2]
    adjb = adj_ref[...].astype(jnp.bfloat16)
    e = jnp.exp2(jnp.minimum(s.astype(jnp.float32), _CLAMP)
                 ).astype(jnp.bfloat16) * adjb
    part = jax.lax.dot_general(
        vt_blk, e, (((1,), (0,)), ((), ())),
        preferred_element_type=jnp.float32)

    @pl.when(j == 0)
    def _init():
        acct_ref[...] = part

    @pl.when(j != 0)
    def _acc():
        acct_ref[...] += part

    @pl.when(j == nsteps - 1)
    def _epilogue():
        acct = acct_ref[...]
        d = acct.shape[0] - 8
        denom = acct[d:d + 1, :]
        acc = (acct[:d, :] * (1.0 / denom)).T
        nps = jnp.concatenate([ns_ref[...], acc], axis=1)
        h = jax.lax.dot_general(
            nps, w1_ref[...], (((1,), (1,)), ((), ())),
            preferred_element_type=jnp.float32) + b1_ref[...]
        h = jnp.maximum(h, 0.0)
        upd = jax.lax.dot_general(
            h, w2_ref[...], (((1,), (1,)), ((), ())),
            preferred_element_type=jnp.float32) + b2_ref[...]
        out_ref[...] = hid_ref[...] + step_ref[0, 0] * upd


@functools.partial(jax.jit, static_argnames=())
def kernel(input_states, hidden_states, adjacency_matrix, Wq, Wk, Wv,
           W1, b1, W2, b2, step_size):
    n, in_sz = input_states.shape
    hid_sz = hidden_states.shape[1]
    d = in_sz + hid_sz
    mlp_h = W1.shape[0]

    ns = jnp.concatenate([input_states, hidden_states], axis=1)
    adj8 = adjacency_matrix.view(jnp.int8)
    Wq = Wq * jnp.float32(_LOG2E)

    r = min(512, n)
    out = pl.pallas_call(
        _fused_kernel,
        grid=(n // r,),
        in_specs=[
            pl.BlockSpec((r, d), lambda j: (j, 0)),
            pl.BlockSpec((r, d),
                         lambda j: (jnp.minimum(j + 1, n // r - 1), 0)),
            pl.BlockSpec((r, n), lambda j: (j, 0)),
            pl.BlockSpec((n, d), lambda j: (0, 0)),
            pl.BlockSpec((n, hid_sz), lambda j: (0, 0)),
            pl.BlockSpec((d, d), lambda j: (0, 0)),
            pl.BlockSpec((d, d), lambda j: (0, 0)),
            pl.BlockSpec((d, d), lambda j: (0, 0)),
            pl.BlockSpec((mlp_h, 2 * d), lambda j: (0, 0)),
            pl.BlockSpec((1, mlp_h), lambda j: (0, 0)),
            pl.BlockSpec((hid_sz, mlp_h), lambda j: (0, 0)),
            pl.BlockSpec((1, hid_sz), lambda j: (0, 0)),
            pl.BlockSpec((1, 1), lambda j: (0, 0)),
        ],
        out_specs=pl.BlockSpec((n, hid_sz), lambda j: (0, 0)),
        out_shape=jax.ShapeDtypeStruct((n, hid_sz), jnp.float32),
        scratch_shapes=[
            pltpu.VMEM((n, d), jnp.bfloat16),
            pltpu.VMEM((2, r, n), jnp.bfloat16),
            pltpu.VMEM((d + 8, n), jnp.float32),
        ],
    )(ns, ns, adj8, ns, hidden_states, Wq, Wk, Wv,
      W1, b1.reshape(1, mlp_h), W2, b2.reshape(1, hid_sz),
      step_size.reshape(1, 1))
    return out


# speedup vs baseline: 1.1199x; 1.1199x over previous
import functools

import jax
import jax.numpy as jnp
from jax.experimental import pallas as pl
from jax.experimental.pallas import tpu as pltpu

_LOG2E = 1.4426950408889634
_CLAMP = 104.0


def _fused_kernel(ns_blk_ref, adj_ref, ns_ref, hid_ref, wq_ref, wk_ref,
                  wv_ref, w1_ref, b1_ref, w2_ref, b2_ref, step_ref,
                  out_ref, q_ref, acct_ref):
    j = pl.program_id(0)
    nsteps = pl.num_programs(0)

    @pl.when(j == 0)
    def _compute_q():
        q_ref[...] = jax.lax.dot_general(
            ns_ref[...], wq_ref[...], (((1,), (1,)), ((), ())),
            preferred_element_type=jnp.float32).astype(jnp.bfloat16)

    ns_blk = ns_blk_ref[...]
    k_blk = jax.lax.dot_general(
        ns_blk, wk_ref[...], (((1,), (1,)), ((), ())),
        preferred_element_type=jnp.float32).astype(jnp.bfloat16)
    v_blk = jax.lax.dot_general(
        ns_blk, wv_ref[...], (((1,), (1,)), ((), ())),
        preferred_element_type=jnp.float32).astype(jnp.bfloat16)
    ones = jnp.ones((8, v_blk.shape[0]), dtype=jnp.bfloat16)
    vt_blk = jnp.concatenate([v_blk.T, ones], axis=0)

    s = jax.lax.dot_general(
        k_blk, q_ref[...], (((1,), (1,)), ((), ())),
        preferred_element_type=jnp.float32)
    adjb = adj_ref[...].astype(jnp.bfloat16)
    e = jnp.exp2(jnp.minimum(s, _CLAMP)).astype(jnp.bfloat16) * adjb
    part = jax.lax.dot_general(
        vt_blk, e, (((1,), (0,)), ((), ())),
        preferred_element_type=jnp.float32)

    @pl.when(j == 0)
    def _init():
        acct_ref[...] = part

    @pl.when(j != 0)
    def _acc():
        acct_ref[...] += part

    @pl.when(j == nsteps - 1)
    def _epilogue():
        acct = acct_ref[...]
        d = acct.shape[0] - 8
        denom = acct[d:d + 1, :]
        acc = (acct[:d, :] * (1.0 / denom)).T
        nps = jnp.concatenate([ns_ref[...], acc], axis=1)
        h = jax.lax.dot_general(
            nps, w1_ref[...], (((1,), (1,)), ((), ())),
            preferred_element_type=jnp.float32) + b1_ref[...]
        h = jnp.maximum(h, 0.0)
        upd = jax.lax.dot_general(
            h, w2_ref[...], (((1,), (1,)), ((), ())),
            preferred_element_type=jnp.float32) + b2_ref[...]
        out_ref[...] = hid_ref[...] + step_ref[0, 0] * upd


@functools.partial(jax.jit, static_argnames=())
def kernel(input_states, hidden_states, adjacency_matrix, Wq, Wk, Wv,
           W1, b1, W2, b2, step_size):
    n, in_sz = input_states.shape
    hid_sz = hidden_states.shape[1]
    d = in_sz + hid_sz
    mlp_h = W1.shape[0]

    ns = jnp.concatenate([input_states, hidden_states], axis=1)
    adj8 = adjacency_matrix.view(jnp.int8)
    Wq = Wq * jnp.float32(_LOG2E)

    r = min(256, n)
    out = pl.pallas_call(
        _fused_kernel,
        grid=(n // r,),
        in_specs=[
            pl.BlockSpec((r, d), lambda j: (j, 0)),
            pl.BlockSpec((r, n), lambda j: (j, 0)),
            pl.BlockSpec((n, d), lambda j: (0, 0)),
            pl.BlockSpec((n, hid_sz), lambda j: (0, 0)),
            pl.BlockSpec((d, d), lambda j: (0, 0)),
            pl.BlockSpec((d, d), lambda j: (0, 0)),
            pl.BlockSpec((d, d), lambda j: (0, 0)),
            pl.BlockSpec((mlp_h, 2 * d), lambda j: (0, 0)),
            pl.BlockSpec((1, mlp_h), lambda j: (0, 0)),
            pl.BlockSpec((hid_sz, mlp_h), lambda j: (0, 0)),
            pl.BlockSpec((1, hid_sz), lambda j: (0, 0)),
            pl.BlockSpec((1, 1), lambda j: (0, 0)),
        ],
        out_specs=pl.BlockSpec((n, hid_sz), lambda j: (0, 0)),
        out_shape=jax.ShapeDtypeStruct((n, hid_sz), jnp.float32),
        scratch_shapes=[
            pltpu.VMEM((n, d), jnp.bfloat16),
            pltpu.VMEM((d + 8, n), jnp.float32),
        ],
    )(ns, adj8, ns, hidden_states, Wq, Wk, Wv,
      W1, b1.reshape(1, mlp_h), W2, b2.reshape(1, hid_sz),
      step_size.reshape(1, 1))
    return out


# speedup vs baseline: 1.1958x; 1.0678x over previous
import functools

import jax
import jax.numpy as jnp
from jax.experimental import pallas as pl
from jax.experimental.pallas import tpu as pltpu

_LOG2E = 1.4426950408889634
_CLAMP = 104.0
_CLAMP_LO = -100.0


def _fused_kernel(ns_blk_ref, adj_ref, ns_ref, hid_ref, wq_ref, wk_ref,
                  wv_ref, w1_ref, b1_ref, w2_ref, b2_ref, step_ref,
                  out_ref, q_ref, acct_ref):
    j = pl.program_id(0)
    nsteps = pl.num_programs(0)

    @pl.when(j == 0)
    def _compute_q():
        q_ref[...] = jax.lax.dot_general(
            ns_ref[...], wq_ref[...], (((1,), (1,)), ((), ())),
            preferred_element_type=jnp.float32).astype(jnp.bfloat16)

    ns_blk = ns_blk_ref[...]
    k_blk = jax.lax.dot_general(
        ns_blk, wk_ref[...], (((1,), (1,)), ((), ())),
        preferred_element_type=jnp.float32).astype(jnp.bfloat16)
    v_blk = jax.lax.dot_general(
        ns_blk, wv_ref[...], (((1,), (1,)), ((), ())),
        preferred_element_type=jnp.float32).astype(jnp.bfloat16)
    ones = jnp.ones((8, v_blk.shape[0]), dtype=jnp.bfloat16)
    vt_blk = jnp.concatenate([v_blk.T, ones], axis=0)

    s = jax.lax.dot_general(
        k_blk, q_ref[...], (((1,), (1,)), ((), ())),
        preferred_element_type=jnp.float32)
    adjb = adj_ref[...].astype(jnp.bfloat16)
    e = jnp.exp2(jnp.clip(s, _CLAMP_LO, _CLAMP)).astype(jnp.bfloat16) * adjb
    part = jax.lax.dot_general(
        vt_blk, e, (((1,), (0,)), ((), ())),
        preferred_element_type=jnp.float32)

    @pl.when(j == 0)
    def _init():
        acct_ref[...] = part

    @pl.when(j != 0)
    def _acc():
        acct_ref[...] += part

    @pl.when(j == nsteps - 1)
    def _epilogue():
        acct = acct_ref[...]
        d = acct.shape[0] - 8
        denom = acct[d:d + 1, :]
        acc = (acct[:d, :] * (1.0 / denom)).T
        nps = jnp.concatenate([ns_ref[...], acc], axis=1)
        h = jax.lax.dot_general(
            nps, w1_ref[...], (((1,), (1,)), ((), ())),
            preferred_element_type=jnp.float32) + b1_ref[...]
        h = jnp.maximum(h, 0.0)
        upd = jax.lax.dot_general(
            h, w2_ref[...], (((1,), (1,)), ((), ())),
            preferred_element_type=jnp.float32) + b2_ref[...]
        out_ref[...] = hid_ref[...] + step_ref[0, 0] * upd


@functools.partial(jax.jit, static_argnames=())
def kernel(input_states, hidden_states, adjacency_matrix, Wq, Wk, Wv,
           W1, b1, W2, b2, step_size):
    n, in_sz = input_states.shape
    hid_sz = hidden_states.shape[1]
    d = in_sz + hid_sz
    mlp_h = W1.shape[0]

    ns = jnp.concatenate([input_states, hidden_states], axis=1)
    adj8 = adjacency_matrix.view(jnp.int8)
    Wq = Wq * jnp.float32(_LOG2E)

    r = min(512, n)
    out = pl.pallas_call(
        _fused_kernel,
        grid=(n // r,),
        in_specs=[
            pl.BlockSpec((r, d), lambda j: (j, 0)),
            pl.BlockSpec((r, n), lambda j: (j, 0)),
            pl.BlockSpec((n, d), lambda j: (0, 0)),
            pl.BlockSpec((n, hid_sz), lambda j: (0, 0)),
            pl.BlockSpec((d, d), lambda j: (0, 0)),
            pl.BlockSpec((d, d), lambda j: (0, 0)),
            pl.BlockSpec((d, d), lambda j: (0, 0)),
            pl.BlockSpec((mlp_h, 2 * d), lambda j: (0, 0)),
            pl.BlockSpec((1, mlp_h), lambda j: (0, 0)),
            pl.BlockSpec((hid_sz, mlp_h), lambda j: (0, 0)),
            pl.BlockSpec((1, hid_sz), lambda j: (0, 0)),
            pl.BlockSpec((1, 1), lambda j: (0, 0)),
        ],
        out_specs=pl.BlockSpec((n, hid_sz), lambda j: (0, 0)),
        out_shape=jax.ShapeDtypeStruct((n, hid_sz), jnp.float32),
        scratch_shapes=[
            pltpu.VMEM((n, d), jnp.bfloat16),
            pltpu.VMEM((d + 8, n), jnp.float32),
        ],
    )(ns, adj8, ns, hidden_states, Wq, Wk, Wv,
      W1, b1.reshape(1, mlp_h), W2, b2.reshape(1, hid_sz),
      step_size.reshape(1, 1))
    return out
